# Initial kernel scaffold; baseline (speedup 1.0000x reference)
import jax
import jax.numpy as jnp
from jax.experimental import pallas as pl


def kernel(x, adj, edge_features, W1_node, W1_edge, a_src1, a_tgt1, a_edge1, W2_node, W2_edge, a_src2, a_tgt2, a_edge2):
    raise NotImplementedError("write your pallas kernel here")


# speedup vs baseline: 345.9807x; 345.9807x over previous
import functools

import jax
import jax.numpy as jnp
from jax import lax
from jax.experimental import pallas as pl

_N = 512
_B = 2
_D = 128
_NHEAD = 4
_HD = _D // _NHEAD
_BI = 16
_PREC = lax.Precision.HIGHEST


def _edge_scores_body(ef_ref, w1e_ref, w2e_ref, ae1_ref, ae2_ref, es_ref):
    row = lax.broadcasted_iota(jnp.int32, (_NHEAD, _D), 0)
    col = lax.broadcasted_iota(jnp.int32, (_NHEAD, _D), 1)
    sel = jnp.where(col // _HD == row, 1.0, 0.0).astype(jnp.float32)
    dn_nt = (((1,), (1,)), ((), ()))
    ef2 = ef_ref[...].reshape(_BI * _N, _D)
    parts = []
    for w_ref, ae_ref in ((w1e_ref, ae1_ref), (w2e_ref, ae2_ref)):
        proj = lax.dot_general(ef2, w_ref[...], dn_nt,
                               precision=lax.Precision.DEFAULT,
                               preferred_element_type=jnp.float32)
        esl = lax.dot_general(sel * ae_ref[...], proj, dn_nt,
                              precision=_PREC,
                              preferred_element_type=jnp.float32)
        parts.append(esl.reshape(_NHEAD, _BI, _N))
    es_ref[...] = jnp.concatenate(parts, axis=0)


def _attn_body(x_ref, mask_ref, es_ref, wn_ref, asrc_ref, atgt_ref, out_ref):
    maskf = mask_ref[...]
    wn = wn_ref[...]
    dn_nt = (((1,), (1,)), ((), ()))
    dn_tn = (((0,), (0,)), ((), ()))
    for b in range(_B):
        xb = x_ref[b]
        h = lax.dot_general(xb, wn, dn_nt,
                            precision=lax.Precision.DEFAULT,
                            preferred_element_type=jnp.float32)
        outs = []
        for hh in range(_NHEAD):
            h_h = h[:, hh * _HD:(hh + 1) * _HD]
            ss = lax.dot_general(h_h, asrc_ref[hh:hh + 1, :], dn_nt,
                                 precision=_PREC,
                                 preferred_element_type=jnp.float32)
            st = lax.dot_general(atgt_ref[hh:hh + 1, :], h_h, dn_nt,
                                 precision=_PREC,
                                 preferred_element_type=jnp.float32)
            sc = ss + st + es_ref[hh]
            z = jnp.where(sc >= 0, sc, 0.2 * sc)
            p = maskf * jnp.exp(z)
            denom = jnp.sum(p, axis=0, keepdims=True)
            attn = p / (denom + 1e-16)
            outs.append(lax.dot_general(attn, h_h, dn_tn,
                                        precision=_PREC,
                                        preferred_element_type=jnp.float32))
        out_ref[b] = jnp.concatenate(outs, axis=1)


@functools.partial(jax.jit, static_argnames=())
def kernel(x, adj, edge_features, W1_node, W1_edge, a_src1, a_tgt1, a_edge1,
           W2_node, W2_edge, a_src2, a_tgt2, a_edge2):
    maskf = (adj != 0).astype(jnp.float32)
    ae1 = a_edge1.reshape(_NHEAD, 1).astype(jnp.float32)
    ae2 = a_edge2.reshape(_NHEAD, 1).astype(jnp.float32)

    es = pl.pallas_call(
        _edge_scores_body,
        grid=(_N // _BI,),
        in_specs=[
            pl.BlockSpec((_BI, _N, _D), lambda i: (i, 0, 0)),
            pl.BlockSpec((_D, _D), lambda i: (0, 0)),
            pl.BlockSpec((_D, _D), lambda i: (0, 0)),
            pl.BlockSpec((_NHEAD, 1), lambda i: (0, 0)),
            pl.BlockSpec((_NHEAD, 1), lambda i: (0, 0)),
        ],
        out_specs=pl.BlockSpec((2 * _NHEAD, _BI, _N), lambda i: (0, i, 0)),
        out_shape=jax.ShapeDtypeStruct((2 * _NHEAD, _N, _N), jnp.float32),
    )(edge_features, W1_edge, W2_edge, ae1, ae2)

    def attn_layer(h_bnd, es_l, wn, a_s, a_t):
        return pl.pallas_call(
            _attn_body,
            out_shape=jax.ShapeDtypeStruct((_B, _N, _D), jnp.float32),
        )(h_bnd, maskf, es_l, wn, a_s, a_t)

    x_bnd = jnp.transpose(x, (1, 0, 2))
    h1 = attn_layer(x_bnd, es[:_NHEAD], W1_node, a_src1, a_tgt1)
    h2 = attn_layer(h1, es[_NHEAD:], W2_node, a_src2, a_tgt2)
    return jnp.transpose(h2, (1, 0, 2))


# speedup vs baseline: 756.3347x; 2.1861x over previous
import functools

import jax
import jax.numpy as jnp
from jax import lax
from jax.experimental import pallas as pl

_N = 512
_B = 2
_D = 128
_NHEAD = 4
_HD = _D // _NHEAD
_BI = 16
_PREC = lax.Precision.HIGHEST


def _edge_scores_body(ef_ref, w1e_ref, w2e_ref, ae1_ref, ae2_ref, es_ref):
    wcat = jnp.concatenate([w1e_ref[...], w2e_ref[...]], axis=0)
    aecat = jnp.concatenate([ae1_ref[...], ae2_ref[...]], axis=0)
    dn_nt = (((1,), (1,)), ((), ()))
    ef2 = ef_ref[...].reshape(_BI * _N, _D)
    proj = lax.dot_general(wcat, ef2, dn_nt,
                           precision=lax.Precision.DEFAULT,
                           preferred_element_type=jnp.float32)
    grouped = proj.reshape(2 * _NHEAD, _HD, _BI * _N).sum(axis=1)
    es_ref[...] = (grouped * aecat).reshape(2 * _NHEAD, _BI, _N)


def _attn_body(x_ref, mask_ref, es_ref, wn_ref, asrc_ref, atgt_ref, out_ref):
    maskf = mask_ref[...]
    wn = wn_ref[...]
    dn_nt = (((1,), (1,)), ((), ()))
    dn_tn = (((0,), (0,)), ((), ()))
    for b in range(_B):
        xb = x_ref[b]
        h = lax.dot_general(xb, wn, dn_nt,
                            precision=lax.Precision.DEFAULT,
                            preferred_element_type=jnp.float32)
        outs = []
        for hh in range(_NHEAD):
            h_h = h[:, hh * _HD:(hh + 1) * _HD]
            ss = lax.dot_general(h_h, asrc_ref[hh:hh + 1, :], dn_nt,
                                 precision=_PREC,
                                 preferred_element_type=jnp.float32)
            st = lax.dot_general(atgt_ref[hh:hh + 1, :], h_h, dn_nt,
                                 precision=_PREC,
                                 preferred_element_type=jnp.float32)
            sc = ss + st + es_ref[hh]
            z = jnp.where(sc >= 0, sc, 0.2 * sc)
            p = maskf * jnp.exp(z)
            denom = jnp.sum(p, axis=0, keepdims=True)
            attn = p / (denom + 1e-16)
            outs.append(lax.dot_general(attn, h_h, dn_tn,
                                        precision=_PREC,
                                        preferred_element_type=jnp.float32))
        out_ref[b] = jnp.concatenate(outs, axis=1)


@functools.partial(jax.jit, static_argnames=())
def kernel(x, adj, edge_features, W1_node, W1_edge, a_src1, a_tgt1, a_edge1,
           W2_node, W2_edge, a_src2, a_tgt2, a_edge2):
    maskf = (adj != 0).astype(jnp.float32)
    ae1 = a_edge1.reshape(_NHEAD, 1).astype(jnp.float32)
    ae2 = a_edge2.reshape(_NHEAD, 1).astype(jnp.float32)

    es = pl.pallas_call(
        _edge_scores_body,
        grid=(_N // _BI,),
        in_specs=[
            pl.BlockSpec((_BI, _N, _D), lambda i: (i, 0, 0)),
            pl.BlockSpec((_D, _D), lambda i: (0, 0)),
            pl.BlockSpec((_D, _D), lambda i: (0, 0)),
            pl.BlockSpec((_NHEAD, 1), lambda i: (0, 0)),
            pl.BlockSpec((_NHEAD, 1), lambda i: (0, 0)),
        ],
        out_specs=pl.BlockSpec((2 * _NHEAD, _BI, _N), lambda i: (0, i, 0)),
        out_shape=jax.ShapeDtypeStruct((2 * _NHEAD, _N, _N), jnp.float32),
    )(edge_features, W1_edge, W2_edge, ae1, ae2)

    def attn_layer(h_bnd, es_l, wn, a_s, a_t):
        return pl.pallas_call(
            _attn_body,
            out_shape=jax.ShapeDtypeStruct((_B, _N, _D), jnp.float32),
        )(h_bnd, maskf, es_l, wn, a_s, a_t)

    x_bnd = jnp.transpose(x, (1, 0, 2))
    h1 = attn_layer(x_bnd, es[:_NHEAD], W1_node, a_src1, a_tgt1)
    h2 = attn_layer(h1, es[_NHEAD:], W2_node, a_src2, a_tgt2)
    return jnp.transpose(h2, (1, 0, 2))


# speedup vs baseline: 797.4593x; 1.0544x over previous
import functools

import jax
import jax.numpy as jnp
from jax import lax
from jax.experimental import pallas as pl

_N = 512
_B = 2
_D = 128
_NHEAD = 4
_HD = _D // _NHEAD
_BI = 32
_PREC = lax.Precision.HIGHEST


def _edge_scores_body(ef_ref, w1e_ref, w2e_ref, ae1_ref, ae2_ref, es_ref):
    wcat = jnp.concatenate([w1e_ref[...], w2e_ref[...]], axis=0)
    aecat = jnp.concatenate([ae1_ref[...], ae2_ref[...]], axis=0)
    dn_nt = (((1,), (1,)), ((), ()))
    ef2 = ef_ref[...].reshape(_BI * _N, _D)
    proj = lax.dot_general(wcat, ef2, dn_nt,
                           precision=lax.Precision.DEFAULT,
                           preferred_element_type=jnp.float32)
    grouped = proj.reshape(2 * _NHEAD, _HD, _BI * _N).sum(axis=1)
    es_ref[...] = (grouped * aecat).reshape(2 * _NHEAD, _BI, _N)


def _attn_body(x_ref, adj_ref, es_ref, wn_ref, asrc_ref, atgt_ref, out_ref):
    maskf = (adj_ref[...] != 0).astype(jnp.float32)
    wn = wn_ref[...]
    dn_nt = (((1,), (1,)), ((), ()))
    dn_tn = (((0,), (0,)), ((), ()))
    for b in range(_B):
        xb = x_ref[:, b, :]
        h = lax.dot_general(xb, wn, dn_nt,
                            precision=lax.Precision.DEFAULT,
                            preferred_element_type=jnp.float32)
        outs = []
        for hh in range(_NHEAD):
            h_h = h[:, hh * _HD:(hh + 1) * _HD]
            ss = lax.dot_general(h_h, asrc_ref[hh:hh + 1, :], dn_nt,
                                 precision=_PREC,
                                 preferred_element_type=jnp.float32)
            st = lax.dot_general(atgt_ref[hh:hh + 1, :], h_h, dn_nt,
                                 precision=_PREC,
                                 preferred_element_type=jnp.float32)
            sc = ss + st + es_ref[hh]
            z = jnp.where(sc >= 0, sc, 0.2 * sc)
            p = maskf * jnp.exp(z)
            denom = jnp.sum(p, axis=0, keepdims=True)
            attn = p / (denom + 1e-16)
            outs.append(lax.dot_general(attn, h_h, dn_tn,
                                        precision=_PREC,
                                        preferred_element_type=jnp.float32))
        out_ref[:, b, :] = jnp.concatenate(outs, axis=1)


@functools.partial(jax.jit, static_argnames=())
def kernel(x, adj, edge_features, W1_node, W1_edge, a_src1, a_tgt1, a_edge1,
           W2_node, W2_edge, a_src2, a_tgt2, a_edge2):
    ae1 = a_edge1.reshape(_NHEAD, 1).astype(jnp.float32)
    ae2 = a_edge2.reshape(_NHEAD, 1).astype(jnp.float32)

    es = pl.pallas_call(
        _edge_scores_body,
        grid=(_N // _BI,),
        in_specs=[
            pl.BlockSpec((_BI, _N, _D), lambda i: (i, 0, 0)),
            pl.BlockSpec((_D, _D), lambda i: (0, 0)),
            pl.BlockSpec((_D, _D), lambda i: (0, 0)),
            pl.BlockSpec((_NHEAD, 1), lambda i: (0, 0)),
            pl.BlockSpec((_NHEAD, 1), lambda i: (0, 0)),
        ],
        out_specs=pl.BlockSpec((2 * _NHEAD, _BI, _N), lambda i: (0, i, 0)),
        out_shape=jax.ShapeDtypeStruct((2 * _NHEAD, _N, _N), jnp.float32),
    )(edge_features, W1_edge, W2_edge, ae1, ae2)

    def attn_layer(h_nbd, es_l, wn, a_s, a_t):
        return pl.pallas_call(
            _attn_body,
            out_shape=jax.ShapeDtypeStruct((_N, _B, _D), jnp.float32),
        )(h_nbd, adj, es_l, wn, a_s, a_t)

    h1 = attn_layer(x, es[:_NHEAD], W1_node, a_src1, a_tgt1)
    return attn_layer(h1, es[_NHEAD:], W2_node, a_src2, a_tgt2)


# speedup vs baseline: 1190.1018x; 1.4924x over previous
import functools

import jax
import jax.numpy as jnp
from jax import lax
from jax.experimental import pallas as pl

_N = 512
_B = 2
_D = 128
_NHEAD = 4
_HD = _D // _NHEAD
_BI = 32
_PREC = lax.Precision.HIGHEST


def _edge_scores_body(ef_ref, w1e_ref, w2e_ref, ae1_ref, ae2_ref, es_ref):
    wcat = jnp.concatenate([w1e_ref[...], w2e_ref[...]], axis=0)
    aecat = jnp.concatenate([ae1_ref[...], ae2_ref[...]], axis=0)
    wb = wcat.astype(jnp.bfloat16).astype(jnp.float32)
    wsum = wb.reshape(2 * _NHEAD, _HD, _D).sum(axis=1)
    w_hi = wsum.astype(jnp.bfloat16).astype(jnp.float32)
    w_lo = wsum - w_hi
    whl = jnp.concatenate([w_hi, w_lo], axis=0).astype(jnp.bfloat16)
    dn_nt = (((1,), (1,)), ((), ()))
    efb = ef_ref[...].reshape(_BI * _N, _D).astype(jnp.bfloat16)
    dots = lax.dot_general(whl, efb, dn_nt,
                           precision=lax.Precision.DEFAULT,
                           preferred_element_type=jnp.float32)
    grouped = dots[:2 * _NHEAD] + dots[2 * _NHEAD:]
    es_ref[...] = (grouped * aecat).reshape(2 * _NHEAD, _BI, _N)


def _attn_body(x_ref, adj_ref, es_ref, wn_ref, asrc_ref, atgt_ref, out_ref):
    maskf = (adj_ref[...] != 0).astype(jnp.float32)
    wn = wn_ref[...]
    dn_nt = (((1,), (1,)), ((), ()))
    dn_tn = (((0,), (0,)), ((), ()))
    for b in range(_B):
        xb = x_ref[:, b, :]
        h = lax.dot_general(xb, wn, dn_nt,
                            precision=lax.Precision.DEFAULT,
                            preferred_element_type=jnp.float32)
        outs = []
        for hh in range(_NHEAD):
            h_h = h[:, hh * _HD:(hh + 1) * _HD]
            ss = lax.dot_general(h_h, asrc_ref[hh:hh + 1, :], dn_nt,
                                 precision=_PREC,
                                 preferred_element_type=jnp.float32)
            st = lax.dot_general(atgt_ref[hh:hh + 1, :], h_h, dn_nt,
                                 precision=_PREC,
                                 preferred_element_type=jnp.float32)
            sc = ss + st + es_ref[hh]
            z = jnp.where(sc >= 0, sc, 0.2 * sc)
            p = maskf * jnp.exp(z)
            denom = jnp.sum(p, axis=0, keepdims=True)
            attn = p / (denom + 1e-16)
            outs.append(lax.dot_general(attn, h_h, dn_tn,
                                        precision=_PREC,
                                        preferred_element_type=jnp.float32))
        out_ref[:, b, :] = jnp.concatenate(outs, axis=1)


@functools.partial(jax.jit, static_argnames=())
def kernel(x, adj, edge_features, W1_node, W1_edge, a_src1, a_tgt1, a_edge1,
           W2_node, W2_edge, a_src2, a_tgt2, a_edge2):
    ae1 = a_edge1.reshape(_NHEAD, 1).astype(jnp.float32)
    ae2 = a_edge2.reshape(_NHEAD, 1).astype(jnp.float32)

    es = pl.pallas_call(
        _edge_scores_body,
        grid=(_N // _BI,),
        in_specs=[
            pl.BlockSpec((_BI, _N, _D), lambda i: (i, 0, 0)),
            pl.BlockSpec((_D, _D), lambda i: (0, 0)),
            pl.BlockSpec((_D, _D), lambda i: (0, 0)),
            pl.BlockSpec((_NHEAD, 1), lambda i: (0, 0)),
            pl.BlockSpec((_NHEAD, 1), lambda i: (0, 0)),
        ],
        out_specs=pl.BlockSpec((2 * _NHEAD, _BI, _N), lambda i: (0, i, 0)),
        out_shape=jax.ShapeDtypeStruct((2 * _NHEAD, _N, _N), jnp.float32),
    )(edge_features, W1_edge, W2_edge, ae1, ae2)

    def attn_layer(h_nbd, es_l, wn, a_s, a_t):
        return pl.pallas_call(
            _attn_body,
            out_shape=jax.ShapeDtypeStruct((_N, _B, _D), jnp.float32),
        )(h_nbd, adj, es_l, wn, a_s, a_t)

    h1 = attn_layer(x, es[:_NHEAD], W1_node, a_src1, a_tgt1)
    return attn_layer(h1, es[_NHEAD:], W2_node, a_src2, a_tgt2)


# speedup vs baseline: 1212.1980x; 1.0186x over previous
import functools

import jax
import jax.numpy as jnp
from jax import lax
from jax.experimental import pallas as pl
from jax.experimental.pallas import tpu as pltpu

_N = 512
_B = 2
_D = 128
_NHEAD = 4
_HD = _D // _NHEAD
_BI = 32
_STEPS = _N // _BI
_PREC = lax.Precision.HIGHEST
_DN_NT = (((1,), (1,)), ((), ()))
_DN_TN = (((0,), (0,)), ((), ()))


def _head_scores(h, a_ref, hh):
    h_h = h[:, hh * _HD:(hh + 1) * _HD]
    return lax.dot_general(h_h, a_ref[hh:hh + 1, :], _DN_NT,
                           precision=_PREC,
                           preferred_element_type=jnp.float32)


def _fused_body(ef_ref, adj_ref, x_ref,
                w1e_ref, w2e_ref, ae1_ref, ae2_ref,
                w1n_ref, as1_ref, at1_ref,
                w2n_ref, as2_ref, at2_ref,
                out_ref,
                es2_s, h1_s, ss1_s, st1_s, num_s, den_s):
    c = pl.program_id(0)

    @pl.when(c == 0)
    def _prologue():
        for b in range(_B):
            xb = x_ref[:, b, :]
            h = lax.dot_general(xb, w1n_ref[...], _DN_NT,
                                precision=lax.Precision.DEFAULT,
                                preferred_element_type=jnp.float32)
            h1_s[b] = h
            for hh in range(_NHEAD):
                bh = b * _NHEAD + hh
                ss = _head_scores(h, as1_ref, hh)
                ss1_s[:, bh:bh + 1] = ss
                st = _head_scores(h, at1_ref, hh)
                st1_s[bh:bh + 1, :] = st.reshape(1, _N)
            num_s[b] = jnp.zeros((_N, _D), jnp.float32)
        den_s[...] = jnp.zeros((_B * _NHEAD, _N), jnp.float32)

    wcat = jnp.concatenate([w1e_ref[...], w2e_ref[...]], axis=0)
    aecat = jnp.concatenate([ae1_ref[...], ae2_ref[...]], axis=0)
    wb = wcat.astype(jnp.bfloat16).astype(jnp.float32)
    wsum = wb.reshape(2 * _NHEAD, _HD, _D).sum(axis=1)
    w_hi = wsum.astype(jnp.bfloat16).astype(jnp.float32)
    w_lo = wsum - w_hi
    whl = jnp.concatenate([w_hi, w_lo], axis=0).astype(jnp.bfloat16)
    efb = ef_ref[...].reshape(_BI * _N, _D).astype(jnp.bfloat16)
    dots = lax.dot_general(whl, efb, _DN_NT,
                           precision=lax.Precision.DEFAULT,
                           preferred_element_type=jnp.float32)
    es = (dots[:2 * _NHEAD] + dots[2 * _NHEAD:]) * aecat
    es1 = es[:_NHEAD].reshape(_NHEAD, _BI, _N)
    es2_s[:, pl.ds(c * _BI, _BI), :] = es[_NHEAD:].reshape(_NHEAD, _BI, _N)

    maskf_blk = (adj_ref[pl.ds(c * _BI, _BI), :] != 0).astype(jnp.float32)
    for b in range(_B):
        h_blk = h1_s[b, pl.ds(c * _BI, _BI), :]
        for hh in range(_NHEAD):
            bh = b * _NHEAD + hh
            ss_blk = ss1_s[pl.ds(c * _BI, _BI), bh:bh + 1]
            st_row = st1_s[bh:bh + 1, :]
            sc = ss_blk + st_row + es1[hh]
            z = jnp.where(sc >= 0, sc, 0.2 * sc)
            p = maskf_blk * jnp.exp(z)
            den_s[bh:bh + 1, :] += jnp.sum(p, axis=0, keepdims=True)
            num_s[b, :, hh * _HD:(hh + 1) * _HD] += lax.dot_general(
                p, h_blk[:, hh * _HD:(hh + 1) * _HD], _DN_TN,
                precision=_PREC, preferred_element_type=jnp.float32)

    @pl.when(c == _STEPS - 1)
    def _epilogue():
        recT = jnp.transpose(1.0 / (den_s[...] + 1e-16))
        maskf = (adj_ref[...] != 0).astype(jnp.float32)
        for b in range(_B):
            h1b = jnp.concatenate(
                [num_s[b, :, hh * _HD:(hh + 1) * _HD]
                 * recT[:, b * _NHEAD + hh:b * _NHEAD + hh + 1]
                 for hh in range(_NHEAD)], axis=1)
            h2 = lax.dot_general(h1b, w2n_ref[...], _DN_NT,
                                 precision=lax.Precision.DEFAULT,
                                 preferred_element_type=jnp.float32)
            outs = []
            for hh in range(_NHEAD):
                h_h = h2[:, hh * _HD:(hh + 1) * _HD]
                ss = _head_scores(h2, as2_ref, hh)
                st = _head_scores(h2, at2_ref, hh).reshape(1, _N)
                sc = ss + st + es2_s[hh]
                z = jnp.where(sc >= 0, sc, 0.2 * sc)
                p = maskf * jnp.exp(z)
                denom = jnp.sum(p, axis=0, keepdims=True)
                attn = p / (denom + 1e-16)
                outs.append(lax.dot_general(attn, h_h, _DN_TN,
                                            precision=_PREC,
                                            preferred_element_type=jnp.float32))
            out_ref[:, b, :] = jnp.concatenate(outs, axis=1)


@functools.partial(jax.jit, static_argnames=())
def kernel(x, adj, edge_features, W1_node, W1_edge, a_src1, a_tgt1, a_edge1,
           W2_node, W2_edge, a_src2, a_tgt2, a_edge2):
    ae1 = a_edge1.reshape(_NHEAD, 1).astype(jnp.float32)
    ae2 = a_edge2.reshape(_NHEAD, 1).astype(jnp.float32)
    full = lambda *shape: pl.BlockSpec(shape, lambda i: (0,) * len(shape))
    return pl.pallas_call(
        _fused_body,
        grid=(_STEPS,),
        in_specs=[
            pl.BlockSpec((_BI, _N, _D), lambda i: (i, 0, 0)),
            full(_N, _N),
            full(_N, _B, _D),
            full(_D, _D), full(_D, _D),
            full(_NHEAD, 1), full(_NHEAD, 1),
            full(_D, _D), full(_NHEAD, _HD), full(_NHEAD, _HD),
            full(_D, _D), full(_NHEAD, _HD), full(_NHEAD, _HD),
        ],
        out_specs=full(_N, _B, _D),
        out_shape=jax.ShapeDtypeStruct((_N, _B, _D), jnp.float32),
        scratch_shapes=[
            pltpu.VMEM((_NHEAD, _N, _N), jnp.float32),
            pltpu.VMEM((_B, _N, _D), jnp.float32),
            pltpu.VMEM((_N, _B * _NHEAD), jnp.float32),
            pltpu.VMEM((_B * _NHEAD, _N), jnp.float32),
            pltpu.VMEM((_B, _N, _D), jnp.float32),
            pltpu.VMEM((_B * _NHEAD, _N), jnp.float32),
        ],
    )(edge_features, adj, x, W1_edge, W2_edge, ae1, ae2,
      W1_node, a_src1, a_tgt1, W2_node, a_src2, a_tgt2)


# speedup vs baseline: 1669.1365x; 1.3770x over previous
import functools

import jax
import jax.numpy as jnp
from jax import lax
from jax.experimental import pallas as pl
from jax.experimental.pallas import tpu as pltpu

_N = 512
_B = 2
_D = 128
_NHEAD = 4
_HD = _D // _NHEAD
_BI = 32
_STEPS = _N // _BI
_PREC = lax.Precision.HIGHEST
_DN_NT = (((1,), (1,)), ((), ()))
_DN_TN = (((0,), (0,)), ((), ()))


def _headvec(a_ref):
    tiled = jnp.tile(a_ref[...], (1, _NHEAD))
    row = lax.broadcasted_iota(jnp.int32, (_NHEAD, _D), 0)
    col = lax.broadcasted_iota(jnp.int32, (_NHEAD, _D), 1)
    return jnp.where(col // _HD == row, tiled, 0.0)


def _blockdiag_rows(m, rows_per_head):
    r = m.shape[0]
    tiled = jnp.tile(m, (_NHEAD, 1))
    row = lax.broadcasted_iota(jnp.int32, (_NHEAD * r, _D), 0)
    col = lax.broadcasted_iota(jnp.int32, (_NHEAD * r, _D), 1)
    return jnp.where(row // r == col // _HD, tiled, 0.0)


def _scores_all(h, a_ref):
    return lax.dot_general(h, _headvec(a_ref), _DN_NT,
                           precision=_PREC,
                           preferred_element_type=jnp.float32)


def _fused_body(ef_ref, adj_ref, x_ref,
                w1e_ref, w2e_ref, ae1_ref, ae2_ref,
                w1n_ref, as1_ref, at1_ref,
                w2n_ref, as2_ref, at2_ref,
                out_ref,
                es2_s, h1_s, ss1_s, st1_s, num_s, den_s):
    c = pl.program_id(0)

    @pl.when(c == 0)
    def _prologue():
        for b in range(_B):
            xb = x_ref[:, b, :]
            h = lax.dot_general(xb, w1n_ref[...], _DN_NT,
                                precision=lax.Precision.DEFAULT,
                                preferred_element_type=jnp.float32)
            h1_s[b] = h
            ss1_s[:, b * _NHEAD:(b + 1) * _NHEAD] = _scores_all(h, as1_ref)
            st1_s[b * _NHEAD:(b + 1) * _NHEAD, :] = jnp.transpose(
                _scores_all(h, at1_ref))
            num_s[b] = jnp.zeros((_N, _D), jnp.float32)
        den_s[...] = jnp.zeros((_B * _NHEAD, _N), jnp.float32)

    wcat = jnp.concatenate([w1e_ref[...], w2e_ref[...]], axis=0)
    aecat = jnp.concatenate([ae1_ref[...], ae2_ref[...]], axis=0)
    wb = wcat.astype(jnp.bfloat16).astype(jnp.float32)
    wsum = wb.reshape(2 * _NHEAD, _HD, _D).sum(axis=1)
    w_hi = wsum.astype(jnp.bfloat16).astype(jnp.float32)
    w_lo = wsum - w_hi
    whl = jnp.concatenate([w_hi, w_lo], axis=0).astype(jnp.bfloat16)
    efb = ef_ref[...].reshape(_BI * _N, _D).astype(jnp.bfloat16)
    dots = lax.dot_general(whl, efb, _DN_NT,
                           precision=lax.Precision.DEFAULT,
                           preferred_element_type=jnp.float32)
    es = (dots[:2 * _NHEAD] + dots[2 * _NHEAD:]) * aecat
    es1 = es[:_NHEAD].reshape(_NHEAD, _BI, _N)
    es2_s[:, pl.ds(c * _BI, _BI), :] = es[_NHEAD:].reshape(_NHEAD, _BI, _N)

    maskf_blk = (adj_ref[pl.ds(c * _BI, _BI), :] != 0).astype(jnp.float32)
    for b in range(_B):
        h_blk = h1_s[b, pl.ds(c * _BI, _BI), :]
        ps = []
        for hh in range(_NHEAD):
            bh = b * _NHEAD + hh
            ss_blk = ss1_s[pl.ds(c * _BI, _BI), bh:bh + 1]
            st_row = st1_s[bh:bh + 1, :]
            sc = ss_blk + st_row + es1[hh]
            z = jnp.where(sc >= 0, sc, 0.2 * sc)
            ps.append(maskf_blk * jnp.exp(z))
        p_cat = jnp.concatenate(ps, axis=0)
        den_s[b * _NHEAD:(b + 1) * _NHEAD, :] += (
            p_cat.reshape(_NHEAD, _BI, _N).sum(axis=1))
        num_s[b] += lax.dot_general(
            p_cat, _blockdiag_rows(h_blk, _BI), _DN_TN,
            precision=_PREC, preferred_element_type=jnp.float32)

    @pl.when(c == _STEPS - 1)
    def _epilogue():
        recT = jnp.transpose(1.0 / (den_s[...] + 1e-16))
        maskf = (adj_ref[...] != 0).astype(jnp.float32)
        for b in range(_B):
            h1b = jnp.concatenate(
                [num_s[b, :, hh * _HD:(hh + 1) * _HD]
                 * recT[:, b * _NHEAD + hh:b * _NHEAD + hh + 1]
                 for hh in range(_NHEAD)], axis=1)
            h2 = lax.dot_general(h1b, w2n_ref[...], _DN_NT,
                                 precision=lax.Precision.DEFAULT,
                                 preferred_element_type=jnp.float32)
            ss2 = _scores_all(h2, as2_ref)
            st2 = jnp.transpose(_scores_all(h2, at2_ref))
            ps = []
            for hh in range(_NHEAD):
                sc = ss2[:, hh:hh + 1] + st2[hh:hh + 1, :] + es2_s[hh]
                z = jnp.where(sc >= 0, sc, 0.2 * sc)
                ps.append(maskf * jnp.exp(z))
            p_cat = jnp.concatenate(ps, axis=0)
            den4 = p_cat.reshape(_NHEAD, _N, _N).sum(axis=1)
            rec4 = 1.0 / (den4 + 1e-16)
            attn_cat = p_cat * jnp.broadcast_to(
                rec4.reshape(_NHEAD, 1, _N),
                (_NHEAD, _N, _N)).reshape(_NHEAD * _N, _N)
            out_ref[:, b, :] = lax.dot_general(
                attn_cat, _blockdiag_rows(h2, _N), _DN_TN,
                precision=_PREC, preferred_element_type=jnp.float32)


@functools.partial(jax.jit, static_argnames=())
def kernel(x, adj, edge_features, W1_node, W1_edge, a_src1, a_tgt1, a_edge1,
           W2_node, W2_edge, a_src2, a_tgt2, a_edge2):
    ae1 = a_edge1.reshape(_NHEAD, 1).astype(jnp.float32)
    ae2 = a_edge2.reshape(_NHEAD, 1).astype(jnp.float32)
    full = lambda *shape: pl.BlockSpec(shape, lambda i: (0,) * len(shape))
    return pl.pallas_call(
        _fused_body,
        grid=(_STEPS,),
        in_specs=[
            pl.BlockSpec((_BI, _N, _D), lambda i: (i, 0, 0)),
            full(_N, _N),
            full(_N, _B, _D),
            full(_D, _D), full(_D, _D),
            full(_NHEAD, 1), full(_NHEAD, 1),
            full(_D, _D), full(_NHEAD, _HD), full(_NHEAD, _HD),
            full(_D, _D), full(_NHEAD, _HD), full(_NHEAD, _HD),
        ],
        out_specs=full(_N, _B, _D),
        out_shape=jax.ShapeDtypeStruct((_N, _B, _D), jnp.float32),
        scratch_shapes=[
            pltpu.VMEM((_NHEAD, _N, _N), jnp.float32),
            pltpu.VMEM((_B, _N, _D), jnp.float32),
            pltpu.VMEM((_N, _B * _NHEAD), jnp.float32),
            pltpu.VMEM((_B * _NHEAD, _N), jnp.float32),
            pltpu.VMEM((_B, _N, _D), jnp.float32),
            pltpu.VMEM((_B * _NHEAD, _N), jnp.float32),
        ],
    )(edge_features, adj, x, W1_edge, W2_edge, ae1, ae2,
      W1_node, a_src1, a_tgt1, W2_node, a_src2, a_tgt2)


# speedup vs baseline: 1750.9319x; 1.0490x over previous
import functools

import jax
import jax.numpy as jnp
from jax import lax
from jax.experimental import pallas as pl
from jax.experimental.pallas import tpu as pltpu

_N = 512
_B = 2
_D = 128
_NHEAD = 4
_HD = _D // _NHEAD
_BI = 64
_STEPS = _N // _BI
_PREC = lax.Precision.HIGHEST
_DN_NT = (((1,), (1,)), ((), ()))
_DN_TN = (((0,), (0,)), ((), ()))


def _headvec(a_ref):
    tiled = jnp.tile(a_ref[...], (1, _NHEAD))
    row = lax.broadcasted_iota(jnp.int32, (_NHEAD, _D), 0)
    col = lax.broadcasted_iota(jnp.int32, (_NHEAD, _D), 1)
    return jnp.where(col // _HD == row, tiled, 0.0)


def _blockdiag_rows(m, rows_per_head):
    r = m.shape[0]
    tiled = jnp.tile(m, (_NHEAD, 1))
    row = lax.broadcasted_iota(jnp.int32, (_NHEAD * r, _D), 0)
    col = lax.broadcasted_iota(jnp.int32, (_NHEAD * r, _D), 1)
    return jnp.where(row // r == col // _HD, tiled, 0.0)


def _scores_all(h, a_ref):
    return lax.dot_general(h, _headvec(a_ref), _DN_NT,
                           precision=_PREC,
                           preferred_element_type=jnp.float32)


def _fused_body(ef_ref, adj_ref, x_ref,
                w1e_ref, w2e_ref, ae1_ref, ae2_ref,
                w1n_ref, as1_ref, at1_ref,
                w2n_ref, as2_ref, at2_ref,
                out_ref,
                es2_s, h1_s, ss1_s, st1_s, num_s, den_s):
    c = pl.program_id(0)

    @pl.when(c == 0)
    def _prologue():
        for b in range(_B):
            xb = x_ref[:, b, :]
            h = lax.dot_general(xb, w1n_ref[...], _DN_NT,
                                precision=lax.Precision.DEFAULT,
                                preferred_element_type=jnp.float32)
            h1_s[b] = h
            ss1_s[:, b * _NHEAD:(b + 1) * _NHEAD] = _scores_all(h, as1_ref)
            st1_s[b * _NHEAD:(b + 1) * _NHEAD, :] = jnp.transpose(
                _scores_all(h, at1_ref))
            num_s[b] = jnp.zeros((_N, _D), jnp.float32)
        den_s[...] = jnp.zeros((_B * _NHEAD, _N), jnp.float32)

    wcat = jnp.concatenate([w1e_ref[...], w2e_ref[...]], axis=0)
    aecat = jnp.concatenate([ae1_ref[...], ae2_ref[...]], axis=0)
    wb = wcat.astype(jnp.bfloat16).astype(jnp.float32)
    wsum = wb.reshape(2 * _NHEAD, _HD, _D).sum(axis=1)
    w_hi = wsum.astype(jnp.bfloat16).astype(jnp.float32)
    w_lo = wsum - w_hi
    whl = jnp.concatenate([w_hi, w_lo], axis=0).astype(jnp.bfloat16)
    efb = ef_ref[...].reshape(_BI * _N, _D).astype(jnp.bfloat16)
    dots = lax.dot_general(whl, efb, _DN_NT,
                           precision=lax.Precision.DEFAULT,
                           preferred_element_type=jnp.float32)
    es = (dots[:2 * _NHEAD] + dots[2 * _NHEAD:]) * aecat
    es1 = es[:_NHEAD].reshape(_NHEAD, _BI, _N)
    es2_s[:, pl.ds(c * _BI, _BI), :] = es[_NHEAD:].reshape(_NHEAD, _BI, _N)

    maskf_blk = (adj_ref[pl.ds(c * _BI, _BI), :] != 0).astype(jnp.float32)
    for b in range(_B):
        h_blk = h1_s[b, pl.ds(c * _BI, _BI), :]
        ps = []
        for hh in range(_NHEAD):
            bh = b * _NHEAD + hh
            ss_blk = ss1_s[pl.ds(c * _BI, _BI), bh:bh + 1]
            st_row = st1_s[bh:bh + 1, :]
            sc = ss_blk + st_row + es1[hh]
            z = jnp.where(sc >= 0, sc, 0.2 * sc)
            ps.append(maskf_blk * jnp.exp(z))
        p_cat = jnp.concatenate(ps, axis=0)
        den_s[b * _NHEAD:(b + 1) * _NHEAD, :] += (
            p_cat.reshape(_NHEAD, _BI, _N).sum(axis=1))
        num_s[b] += lax.dot_general(
            p_cat, _blockdiag_rows(h_blk, _BI), _DN_TN,
            precision=_PREC, preferred_element_type=jnp.float32)

    @pl.when(c == _STEPS - 1)
    def _epilogue():
        recT = jnp.transpose(1.0 / (den_s[...] + 1e-16))
        maskf = (adj_ref[...] != 0).astype(jnp.float32)
        for b in range(_B):
            h1b = jnp.concatenate(
                [num_s[b, :, hh * _HD:(hh + 1) * _HD]
                 * recT[:, b * _NHEAD + hh:b * _NHEAD + hh + 1]
                 for hh in range(_NHEAD)], axis=1)
            h2 = lax.dot_general(h1b, w2n_ref[...], _DN_NT,
                                 precision=lax.Precision.DEFAULT,
                                 preferred_element_type=jnp.float32)
            ss2 = _scores_all(h2, as2_ref)
            st2 = jnp.transpose(_scores_all(h2, at2_ref))
            ps = []
            for hh in range(_NHEAD):
                sc = ss2[:, hh:hh + 1] + st2[hh:hh + 1, :] + es2_s[hh]
                z = jnp.where(sc >= 0, sc, 0.2 * sc)
                ps.append(maskf * jnp.exp(z))
            p_cat = jnp.concatenate(ps, axis=0)
            den4 = p_cat.reshape(_NHEAD, _N, _N).sum(axis=1)
            rec4 = 1.0 / (den4 + 1e-16)
            attn_cat = p_cat * jnp.broadcast_to(
                rec4.reshape(_NHEAD, 1, _N),
                (_NHEAD, _N, _N)).reshape(_NHEAD * _N, _N)
            out_ref[:, b, :] = lax.dot_general(
                attn_cat, _blockdiag_rows(h2, _N), _DN_TN,
                precision=_PREC, preferred_element_type=jnp.float32)


@functools.partial(jax.jit, static_argnames=())
def kernel(x, adj, edge_features, W1_node, W1_edge, a_src1, a_tgt1, a_edge1,
           W2_node, W2_edge, a_src2, a_tgt2, a_edge2):
    ae1 = a_edge1.reshape(_NHEAD, 1).astype(jnp.float32)
    ae2 = a_edge2.reshape(_NHEAD, 1).astype(jnp.float32)
    full = lambda *shape: pl.BlockSpec(shape, lambda i: (0,) * len(shape))
    return pl.pallas_call(
        _fused_body,
        grid=(_STEPS,),
        in_specs=[
            pl.BlockSpec((_BI, _N, _D), lambda i: (i, 0, 0)),
            full(_N, _N),
            full(_N, _B, _D),
            full(_D, _D), full(_D, _D),
            full(_NHEAD, 1), full(_NHEAD, 1),
            full(_D, _D), full(_NHEAD, _HD), full(_NHEAD, _HD),
            full(_D, _D), full(_NHEAD, _HD), full(_NHEAD, _HD),
        ],
        out_specs=full(_N, _B, _D),
        out_shape=jax.ShapeDtypeStruct((_N, _B, _D), jnp.float32),
        scratch_shapes=[
            pltpu.VMEM((_NHEAD, _N, _N), jnp.float32),
            pltpu.VMEM((_B, _N, _D), jnp.float32),
            pltpu.VMEM((_N, _B * _NHEAD), jnp.float32),
            pltpu.VMEM((_B * _NHEAD, _N), jnp.float32),
            pltpu.VMEM((_B, _N, _D), jnp.float32),
            pltpu.VMEM((_B * _NHEAD, _N), jnp.float32),
        ],
    )(edge_features, adj, x, W1_edge, W2_edge, ae1, ae2,
      W1_node, a_src1, a_tgt1, W2_node, a_src2, a_tgt2)


# speedup vs baseline: 1852.7633x; 1.0582x over previous
import functools

import jax
import jax.numpy as jnp
from jax import lax
from jax.experimental import pallas as pl
from jax.experimental.pallas import tpu as pltpu

_N = 512
_B = 2
_D = 128
_NHEAD = 4
_HD = _D // _NHEAD
_BI = 64
_STEPS = _N // _BI
_PREC = lax.Precision.HIGHEST
_DN_NT = (((1,), (1,)), ((), ()))
_DN_TN = (((0,), (0,)), ((), ()))


def _headvec(a_ref):
    tiled = jnp.tile(a_ref[...], (1, _NHEAD))
    row = lax.broadcasted_iota(jnp.int32, (_NHEAD, _D), 0)
    col = lax.broadcasted_iota(jnp.int32, (_NHEAD, _D), 1)
    return jnp.where(col // _HD == row, tiled, 0.0)


def _blockdiag_rows(m, rows_per_head):
    r = m.shape[0]
    tiled = jnp.tile(m, (_NHEAD, 1))
    row = lax.broadcasted_iota(jnp.int32, (_NHEAD * r, _D), 0)
    col = lax.broadcasted_iota(jnp.int32, (_NHEAD * r, _D), 1)
    return jnp.where(row // r == col // _HD, tiled, 0.0)


def _scores_all(h, a_ref):
    return lax.dot_general(h, _headvec(a_ref), _DN_NT,
                           precision=_PREC,
                           preferred_element_type=jnp.float32)


def _fused_body(ef_ref, adj_ref, x_ref,
                w1e_ref, w2e_ref, ae1_ref, ae2_ref,
                w1n_ref, as1_ref, at1_ref,
                w2n_ref, as2_ref, at2_ref,
                out_ref,
                es2_s, h1_s, ss1_s, st1_s, num_s, den_s):
    c = pl.program_id(0)

    @pl.when(c == 0)
    def _prologue():
        for b in range(_B):
            xb = x_ref[:, b, :]
            h = lax.dot_general(xb, w1n_ref[...], _DN_NT,
                                precision=lax.Precision.DEFAULT,
                                preferred_element_type=jnp.float32)
            h1_s[b] = h
            ss1_s[:, b * _NHEAD:(b + 1) * _NHEAD] = _scores_all(h, as1_ref)
            st1_s[b * _NHEAD:(b + 1) * _NHEAD, :] = jnp.transpose(
                _scores_all(h, at1_ref))
            num_s[b] = jnp.zeros((_N, _D), jnp.float32)
        den_s[...] = jnp.zeros((_B * _NHEAD, _N), jnp.float32)

    wcat = jnp.concatenate([w1e_ref[...], w2e_ref[...]], axis=0)
    aecat = jnp.concatenate([ae1_ref[...], ae2_ref[...]], axis=0)
    wb = wcat.astype(jnp.bfloat16).astype(jnp.float32)
    wsum = wb.reshape(2 * _NHEAD, _HD, _D).sum(axis=1)
    w_hi = wsum.astype(jnp.bfloat16).astype(jnp.float32)
    w_lo = wsum - w_hi
    whl = jnp.concatenate([w_hi, w_lo], axis=0).astype(jnp.bfloat16)
    efb = ef_ref[...].reshape(_BI * _N, _D).astype(jnp.bfloat16)
    dots = lax.dot_general(whl, efb, _DN_NT,
                           precision=lax.Precision.DEFAULT,
                           preferred_element_type=jnp.float32)
    es = (dots[:2 * _NHEAD] + dots[2 * _NHEAD:]) * aecat
    es1 = es[:_NHEAD].reshape(_NHEAD, _BI, _N)
    es2_s[:, pl.ds(c * _BI, _BI), :] = es[_NHEAD:].reshape(_NHEAD, _BI, _N)

    maskf_blk = (adj_ref[pl.ds(c * _BI, _BI), :] != 0).astype(jnp.float32)
    for b in range(_B):
        h_blk = h1_s[b, pl.ds(c * _BI, _BI), :]
        ps = []
        for hh in range(_NHEAD):
            bh = b * _NHEAD + hh
            ss_blk = ss1_s[pl.ds(c * _BI, _BI), bh:bh + 1]
            st_row = st1_s[bh:bh + 1, :]
            sc = ss_blk + st_row + es1[hh]
            z = jnp.where(sc >= 0, sc, 0.2 * sc)
            ps.append(maskf_blk * jnp.exp(z))
        p_cat = jnp.concatenate(ps, axis=0)
        den_s[b * _NHEAD:(b + 1) * _NHEAD, :] += (
            p_cat.reshape(_NHEAD, _BI, _N).sum(axis=1))
        num_s[b] += lax.dot_general(
            p_cat, _blockdiag_rows(h_blk, _BI), _DN_TN,
            precision=_PREC, preferred_element_type=jnp.float32)

    @pl.when(c == _STEPS - 1)
    def _epilogue():
        recT = jnp.transpose(1.0 / (den_s[...] + 1e-16))
        maskf = (adj_ref[...] != 0).astype(jnp.float32)
        for b in range(_B):
            h1b = jnp.concatenate(
                [num_s[b, :, hh * _HD:(hh + 1) * _HD]
                 * recT[:, b * _NHEAD + hh:b * _NHEAD + hh + 1]
                 for hh in range(_NHEAD)], axis=1)
            h2 = lax.dot_general(h1b, w2n_ref[...], _DN_NT,
                                 precision=lax.Precision.DEFAULT,
                                 preferred_element_type=jnp.float32)
            ss2 = _scores_all(h2, as2_ref)
            st2 = jnp.transpose(_scores_all(h2, at2_ref))
            ps = []
            for hh in range(_NHEAD):
                sc = ss2[:, hh:hh + 1] + st2[hh:hh + 1, :] + es2_s[hh]
                z = jnp.where(sc >= 0, sc, 0.2 * sc)
                ps.append(maskf * jnp.exp(z))
            p_cat = jnp.concatenate(ps, axis=0)
            den4 = p_cat.reshape(_NHEAD, _N, _N).sum(axis=1)
            rec4 = 1.0 / (den4 + 1e-16)
            attn_cat = p_cat * jnp.broadcast_to(
                rec4.reshape(_NHEAD, 1, _N),
                (_NHEAD, _N, _N)).reshape(_NHEAD * _N, _N)
            hbd = _blockdiag_rows(h2, _N)
            a_hi = attn_cat.astype(jnp.bfloat16)
            a_lo = (attn_cat - a_hi.astype(jnp.float32)).astype(jnp.bfloat16)
            h_hi = hbd.astype(jnp.bfloat16)
            h_lo = (hbd - h_hi.astype(jnp.float32)).astype(jnp.bfloat16)
            out_b = lax.dot_general(a_hi, h_hi, _DN_TN,
                                    precision=lax.Precision.DEFAULT,
                                    preferred_element_type=jnp.float32)
            out_b += lax.dot_general(a_hi, h_lo, _DN_TN,
                                     precision=lax.Precision.DEFAULT,
                                     preferred_element_type=jnp.float32)
            out_b += lax.dot_general(a_lo, h_hi, _DN_TN,
                                     precision=lax.Precision.DEFAULT,
                                     preferred_element_type=jnp.float32)
            out_ref[:, b, :] = out_b


@functools.partial(jax.jit, static_argnames=())
def kernel(x, adj, edge_features, W1_node, W1_edge, a_src1, a_tgt1, a_edge1,
           W2_node, W2_edge, a_src2, a_tgt2, a_edge2):
    ae1 = a_edge1.reshape(_NHEAD, 1).astype(jnp.float32)
    ae2 = a_edge2.reshape(_NHEAD, 1).astype(jnp.float32)
    full = lambda *shape: pl.BlockSpec(shape, lambda i: (0,) * len(shape))
    return pl.pallas_call(
        _fused_body,
        grid=(_STEPS,),
        in_specs=[
            pl.BlockSpec((_BI, _N, _D), lambda i: (i, 0, 0)),
            full(_N, _N),
            full(_N, _B, _D),
            full(_D, _D), full(_D, _D),
            full(_NHEAD, 1), full(_NHEAD, 1),
            full(_D, _D), full(_NHEAD, _HD), full(_NHEAD, _HD),
            full(_D, _D), full(_NHEAD, _HD), full(_NHEAD, _HD),
        ],
        out_specs=full(_N, _B, _D),
        out_shape=jax.ShapeDtypeStruct((_N, _B, _D), jnp.float32),
        scratch_shapes=[
            pltpu.VMEM((_NHEAD, _N, _N), jnp.float32),
            pltpu.VMEM((_B, _N, _D), jnp.float32),
            pltpu.VMEM((_N, _B * _NHEAD), jnp.float32),
            pltpu.VMEM((_B * _NHEAD, _N), jnp.float32),
            pltpu.VMEM((_B, _N, _D), jnp.float32),
            pltpu.VMEM((_B * _NHEAD, _N), jnp.float32),
        ],
    )(edge_features, adj, x, W1_edge, W2_edge, ae1, ae2,
      W1_node, a_src1, a_tgt1, W2_node, a_src2, a_tgt2)


# speedup vs baseline: 1869.6922x; 1.0091x over previous
import functools

import jax
import jax.numpy as jnp
from jax import lax
from jax.experimental import pallas as pl
from jax.experimental.pallas import tpu as pltpu

_N = 512
_B = 2
_D = 128
_NHEAD = 4
_HD = _D // _NHEAD
_BI = 64
_STEPS = _N // _BI
_PREC = lax.Precision.HIGHEST
_DN_NT = (((1,), (1,)), ((), ()))
_DN_TN = (((0,), (0,)), ((), ()))


def _headvec(a_ref):
    tiled = jnp.tile(a_ref[...], (1, _NHEAD))
    row = lax.broadcasted_iota(jnp.int32, (_NHEAD, _D), 0)
    col = lax.broadcasted_iota(jnp.int32, (_NHEAD, _D), 1)
    return jnp.where(col // _HD == row, tiled, 0.0)


def _blockdiag_rows(m, rows_per_head):
    r = m.shape[0]
    tiled = jnp.tile(m, (_NHEAD, 1))
    row = lax.broadcasted_iota(jnp.int32, (_NHEAD * r, _D), 0)
    col = lax.broadcasted_iota(jnp.int32, (_NHEAD * r, _D), 1)
    return jnp.where(row // r == col // _HD, tiled, 0.0)


def _scores_all(h, a_ref):
    return lax.dot_general(h, _headvec(a_ref), _DN_NT,
                           precision=_PREC,
                           preferred_element_type=jnp.float32)


def _fused_body(ef_ref, adj_ref, x_ref,
                w1e_ref, w2e_ref, ae1_ref, ae2_ref,
                w1n_ref, as1_ref, at1_ref,
                w2n_ref, as2_ref, at2_ref,
                out_ref,
                es2_s, h1_s, ss1_s, st1_s, num_s, den_s):
    c = pl.program_id(0)

    @pl.when(c == 0)
    def _prologue():
        for b in range(_B):
            xb = x_ref[:, b, :]
            h = lax.dot_general(xb, w1n_ref[...], _DN_NT,
                                precision=lax.Precision.DEFAULT,
                                preferred_element_type=jnp.float32)
            h1_s[b] = h
            ss1_s[:, b * _NHEAD:(b + 1) * _NHEAD] = _scores_all(h, as1_ref)
            st1_s[b * _NHEAD:(b + 1) * _NHEAD, :] = jnp.transpose(
                _scores_all(h, at1_ref))
            num_s[b] = jnp.zeros((_N, _D), jnp.float32)
        den_s[...] = jnp.zeros((_B * _NHEAD, _N), jnp.float32)

    wcat = jnp.concatenate([w1e_ref[...], w2e_ref[...]], axis=0)
    aecat = jnp.concatenate([ae1_ref[...], ae2_ref[...]], axis=0)
    wb = wcat.astype(jnp.bfloat16).astype(jnp.float32)
    wsum = wb.reshape(2 * _NHEAD, _HD, _D).sum(axis=1)
    w_hi = wsum.astype(jnp.bfloat16).astype(jnp.float32)
    w_lo = wsum - w_hi
    whl = jnp.concatenate([w_hi, w_lo], axis=0).astype(jnp.bfloat16)
    efb = ef_ref[...].reshape(_BI * _N, _D).astype(jnp.bfloat16)
    dots = lax.dot_general(whl, efb, _DN_NT,
                           precision=lax.Precision.DEFAULT,
                           preferred_element_type=jnp.float32)
    es = (dots[:2 * _NHEAD] + dots[2 * _NHEAD:]) * aecat
    es1 = es[:_NHEAD].reshape(_NHEAD, _BI, _N)
    es2_s[:, pl.ds(c * _BI, _BI), :] = es[_NHEAD:].reshape(_NHEAD, _BI, _N)

    maskf_blk = (adj_ref[pl.ds(c * _BI, _BI), :] != 0).astype(jnp.float32)
    for b in range(_B):
        h_blk = h1_s[b, pl.ds(c * _BI, _BI), :]
        ps = []
        for hh in range(_NHEAD):
            bh = b * _NHEAD + hh
            ss_blk = ss1_s[pl.ds(c * _BI, _BI), bh:bh + 1]
            st_row = st1_s[bh:bh + 1, :]
            sc = ss_blk + st_row + es1[hh]
            z = jnp.where(sc >= 0, sc, 0.2 * sc)
            ps.append(maskf_blk * jnp.exp(z))
        p_cat = jnp.concatenate(ps, axis=0)
        den_s[b * _NHEAD:(b + 1) * _NHEAD, :] += (
            p_cat.reshape(_NHEAD, _BI, _N).sum(axis=1))
        hbd = _blockdiag_rows(h_blk, _BI)
        p_hi = p_cat.astype(jnp.bfloat16)
        p_lo = (p_cat - p_hi.astype(jnp.float32)).astype(jnp.bfloat16)
        g_hi = hbd.astype(jnp.bfloat16)
        g_lo = (hbd - g_hi.astype(jnp.float32)).astype(jnp.bfloat16)
        acc = lax.dot_general(p_hi, g_hi, _DN_TN,
                              precision=lax.Precision.DEFAULT,
                              preferred_element_type=jnp.float32)
        acc += lax.dot_general(p_hi, g_lo, _DN_TN,
                               precision=lax.Precision.DEFAULT,
                               preferred_element_type=jnp.float32)
        acc += lax.dot_general(p_lo, g_hi, _DN_TN,
                               precision=lax.Precision.DEFAULT,
                               preferred_element_type=jnp.float32)
        num_s[b] += acc

    @pl.when(c == _STEPS - 1)
    def _epilogue():
        recT = jnp.transpose(1.0 / (den_s[...] + 1e-16))
        maskf = (adj_ref[...] != 0).astype(jnp.float32)
        for b in range(_B):
            h1b = jnp.concatenate(
                [num_s[b, :, hh * _HD:(hh + 1) * _HD]
                 * recT[:, b * _NHEAD + hh:b * _NHEAD + hh + 1]
                 for hh in range(_NHEAD)], axis=1)
            h2 = lax.dot_general(h1b, w2n_ref[...], _DN_NT,
                                 precision=lax.Precision.DEFAULT,
                                 preferred_element_type=jnp.float32)
            ss2 = _scores_all(h2, as2_ref)
            st2 = jnp.transpose(_scores_all(h2, at2_ref))
            ps = []
            for hh in range(_NHEAD):
                sc = ss2[:, hh:hh + 1] + st2[hh:hh + 1, :] + es2_s[hh]
                z = jnp.where(sc >= 0, sc, 0.2 * sc)
                ps.append(maskf * jnp.exp(z))
            p_cat = jnp.concatenate(ps, axis=0)
            den4 = p_cat.reshape(_NHEAD, _N, _N).sum(axis=1)
            rec4 = 1.0 / (den4 + 1e-16)
            attn_cat = p_cat * jnp.broadcast_to(
                rec4.reshape(_NHEAD, 1, _N),
                (_NHEAD, _N, _N)).reshape(_NHEAD * _N, _N)
            hbd = _blockdiag_rows(h2, _N)
            a_hi = attn_cat.astype(jnp.bfloat16)
            a_lo = (attn_cat - a_hi.astype(jnp.float32)).astype(jnp.bfloat16)
            h_hi = hbd.astype(jnp.bfloat16)
            h_lo = (hbd - h_hi.astype(jnp.float32)).astype(jnp.bfloat16)
            out_b = lax.dot_general(a_hi, h_hi, _DN_TN,
                                    precision=lax.Precision.DEFAULT,
                                    preferred_element_type=jnp.float32)
            out_b += lax.dot_general(a_hi, h_lo, _DN_TN,
                                     precision=lax.Precision.DEFAULT,
                                     preferred_element_type=jnp.float32)
            out_b += lax.dot_general(a_lo, h_hi, _DN_TN,
                                     precision=lax.Precision.DEFAULT,
                                     preferred_element_type=jnp.float32)
            out_ref[:, b, :] = out_b


@functools.partial(jax.jit, static_argnames=())
def kernel(x, adj, edge_features, W1_node, W1_edge, a_src1, a_tgt1, a_edge1,
           W2_node, W2_edge, a_src2, a_tgt2, a_edge2):
    ae1 = a_edge1.reshape(_NHEAD, 1).astype(jnp.float32)
    ae2 = a_edge2.reshape(_NHEAD, 1).astype(jnp.float32)
    full = lambda *shape: pl.BlockSpec(shape, lambda i: (0,) * len(shape))
    return pl.pallas_call(
        _fused_body,
        grid=(_STEPS,),
        in_specs=[
            pl.BlockSpec((_BI, _N, _D), lambda i: (i, 0, 0)),
            full(_N, _N),
            full(_N, _B, _D),
            full(_D, _D), full(_D, _D),
            full(_NHEAD, 1), full(_NHEAD, 1),
            full(_D, _D), full(_NHEAD, _HD), full(_NHEAD, _HD),
            full(_D, _D), full(_NHEAD, _HD), full(_NHEAD, _HD),
        ],
        out_specs=full(_N, _B, _D),
        out_shape=jax.ShapeDtypeStruct((_N, _B, _D), jnp.float32),
        scratch_shapes=[
            pltpu.VMEM((_NHEAD, _N, _N), jnp.float32),
            pltpu.VMEM((_B, _N, _D), jnp.float32),
            pltpu.VMEM((_N, _B * _NHEAD), jnp.float32),
            pltpu.VMEM((_B * _NHEAD, _N), jnp.float32),
            pltpu.VMEM((_B, _N, _D), jnp.float32),
            pltpu.VMEM((_B * _NHEAD, _N), jnp.float32),
        ],
    )(edge_features, adj, x, W1_edge, W2_edge, ae1, ae2,
      W1_node, a_src1, a_tgt1, W2_node, a_src2, a_tgt2)
